# baseline (speedup 1.0000x reference)
import functools

import jax
import jax.numpy as jnp
from jax import lax
from jax.experimental import pallas as pl
from jax.experimental.pallas import tpu as pltpu
from jax.experimental.pallas import tpu_sc as plsc

_CHUNK = 128
_SG = 8
_NBUF = 3
_NW = 32
_LANES = 16


def _make_gather(dim: int, b: int, s: int):
    rows_w = b // _NW
    tok_w = rows_w * s
    chunks = tok_w // _CHUNK
    ngroups = chunks // _SG
    rows_per_g = _SG * _CHUNK
    mesh = plsc.VectorSubcoreMesh(core_axis_name="c", subcore_axis_name="s")

    @functools.partial(
        pl.kernel,
        mesh=mesh,
        out_type=jax.ShapeDtypeStruct((b * s, dim), jnp.float32),
        compiler_params=pltpu.CompilerParams(use_tc_tiling_on_sc=False),
        scratch_types=[
            pltpu.VMEM((rows_w, s), jnp.int32),
            pltpu.VMEM((tok_w,), jnp.int32),
            *[pltpu.VMEM((rows_per_g, dim), jnp.float32) for _ in range(_NBUF)],
            *[pltpu.SemaphoreType.DMA for _ in range(2 * _NBUF)],
        ],
    )
    def gather_kernel(idx_hbm, table_hbm, out_hbm, idx_v, idx_f, *scr):
        stage = scr[:_NBUF]
        gsem = scr[_NBUF:2 * _NBUF]
        wsem = scr[2 * _NBUF:]
        wid = lax.axis_index("s") * 2 + lax.axis_index("c")
        base = wid * tok_w
        pltpu.sync_copy(idx_hbm.at[pl.ds(wid * rows_w, rows_w)], idx_v)

        def repack(r, carry):
            idx_f[pl.ds(r * s, _LANES)] = idx_v[r, pl.ds(0, _LANES)]
            idx_f[pl.ds(r * s + (s - _LANES), _LANES)] = idx_v[
                r, pl.ds(s - _LANES, _LANES)
            ]
            return carry

        lax.fori_loop(0, rows_w, repack, 0)

        def fire(g, p):
            return [
                pltpu.async_copy(
                    table_hbm.at[idx_f.at[pl.ds((g * _SG + i) * _CHUNK, _CHUNK)]],
                    stage[p].at[pl.ds(i * _CHUNK, _CHUNK)],
                    gsem[p],
                )
                for i in range(_SG)
            ]

        pending = [None] * _NBUF
        writes = [None] * _NBUF
        pending[0] = fire(0, 0)
        for g in range(ngroups):
            p = g % _NBUF
            if g + 1 < ngroups:
                q = (g + 1) % _NBUF
                if writes[q] is not None:
                    writes[q].wait()
                    writes[q] = None
                pending[q] = fire(g + 1, q)
            for c in pending[p]:
                c.wait()
            writes[p] = pltpu.async_copy(
                stage[p],
                out_hbm.at[pl.ds(base + g * rows_per_g, rows_per_g)],
                wsem[p],
            )
        for w in writes:
            if w is not None:
                w.wait()

    return gather_kernel


def kernel(token_ids, weight):
    b, s = token_ids.shape
    _, dim = weight.shape
    ids = token_ids.astype(jnp.int32)
    out = _make_gather(dim, b, s)(ids, weight)
    return out.reshape(b, s, dim)


# speedup vs baseline: 1.0017x; 1.0017x over previous
import functools

import jax
import jax.numpy as jnp
from jax import lax
from jax.experimental import pallas as pl
from jax.experimental.pallas import tpu as pltpu
from jax.experimental.pallas import tpu_sc as plsc

_GROWS = 32
_NBUF = 3
_NW = 32


def _make_gather(dim: int, b: int, s: int):
    rows_w = b // _NW
    ngroups = rows_w // _GROWS
    mesh = plsc.VectorSubcoreMesh(core_axis_name="c", subcore_axis_name="s")

    @functools.partial(
        pl.kernel,
        mesh=mesh,
        out_type=jax.ShapeDtypeStruct((b, s, dim), jnp.float32),
        compiler_params=pltpu.CompilerParams(use_tc_tiling_on_sc=False),
        scratch_types=[
            pltpu.VMEM((rows_w, s), jnp.int32),
            *[pltpu.VMEM((_GROWS, s, dim), jnp.float32) for _ in range(_NBUF)],
            *[pltpu.SemaphoreType.DMA for _ in range(2 * _NBUF)],
        ],
    )
    def gather_kernel(idx_hbm, table_hbm, out_hbm, idx_v, *scr):
        stage = scr[:_NBUF]
        gsem = scr[_NBUF:2 * _NBUF]
        wsem = scr[2 * _NBUF:]
        wid = lax.axis_index("s") * 2 + lax.axis_index("c")
        base = wid * rows_w
        pltpu.sync_copy(idx_hbm.at[pl.ds(base, rows_w)], idx_v)

        def fire(g, p):
            return [
                pltpu.async_copy(
                    table_hbm.at[idx_v.at[g * _GROWS + r]],
                    stage[p].at[r],
                    gsem[p],
                )
                for r in range(_GROWS)
            ]

        pending = [None] * _NBUF
        writes = [None] * _NBUF
        pending[0] = fire(0, 0)
        for g in range(ngroups):
            p = g % _NBUF
            if g + 1 < ngroups:
                q = (g + 1) % _NBUF
                if writes[q] is not None:
                    writes[q].wait()
                    writes[q] = None
                pending[q] = fire(g + 1, q)
            for c in pending[p]:
                c.wait()
            writes[p] = pltpu.async_copy(
                stage[p],
                out_hbm.at[pl.ds(base + g * _GROWS, _GROWS)],
                wsem[p],
            )
        for w in writes:
            if w is not None:
                w.wait()

    return gather_kernel


def kernel(token_ids, weight):
    b, s = token_ids.shape
    _, dim = weight.shape
    ids = token_ids.astype(jnp.int32)
    return _make_gather(dim, b, s)(ids, weight)


# speedup vs baseline: 1.0783x; 1.0765x over previous
import functools

import jax
import jax.numpy as jnp
from jax import lax
from jax.experimental import pallas as pl
from jax.experimental.pallas import tpu as pltpu
from jax.experimental.pallas import tpu_sc as plsc

_GROWS = 8
_SLAB = 128
_NW = 32


def _make_gather(dim: int, b: int, s: int):
    rows_w = b // _NW
    npairs = rows_w // (2 * _GROWS)
    pairs_per_slab = _SLAB // (2 * _GROWS)
    mesh = plsc.VectorSubcoreMesh(core_axis_name="c", subcore_axis_name="s")

    @functools.partial(
        pl.kernel,
        mesh=mesh,
        out_type=jax.ShapeDtypeStruct((b, s, dim), jnp.float32),
        compiler_params=pltpu.CompilerParams(use_tc_tiling_on_sc=True),
        scratch_types=[
            pltpu.VMEM((_SLAB, s), jnp.int32),
            pltpu.VMEM((_GROWS, s, dim), jnp.float32),
            pltpu.VMEM((_GROWS, s, dim), jnp.float32),
            pltpu.SemaphoreType.DMA,
            pltpu.SemaphoreType.DMA,
            pltpu.SemaphoreType.DMA,
            pltpu.SemaphoreType.DMA,
        ],
    )
    def gather_kernel(idx_hbm, table_hbm, out_hbm, idx_v, st0, st1, g0, g1, w0, w1):
        stage = (st0, st1)
        gsem = (g0, g1)
        wsem = (w0, w1)
        wid = lax.axis_index("s") * 2 + lax.axis_index("c")
        base = wid * rows_w

        def pair(g2, carry):
            @pl.when(g2 % pairs_per_slab == 0)
            def _():
                pltpu.sync_copy(
                    idx_hbm.at[pl.ds(base + (g2 // pairs_per_slab) * _SLAB, _SLAB)],
                    idx_v,
                )

            for p in range(2):
                grow0 = (g2 * 2 + p) * _GROWS
                srow0 = (g2 % pairs_per_slab) * 2 * _GROWS + p * _GROWS

                @pl.when(g2 > 0)
                def _():
                    pltpu.make_async_copy(
                        out_hbm.at[pl.ds(0, _GROWS)], stage[p], wsem[p]
                    ).wait()

                def row(r, c):
                    va = idx_v[srow0 + r, pl.ds(0, 16)]
                    vb = idx_v[srow0 + r, pl.ds(s - 16, 16)]
                    for t in range(s):
                        rid = va[t] if t < 16 else vb[t - (s - 16)]
                        pltpu.async_copy(
                            table_hbm.at[pl.ds(rid, 1)],
                            stage[p].at[r].at[pl.ds(t, 1)],
                            gsem[p],
                        )

                    @pl.when(r > 0)
                    def _():
                        pltpu.make_async_copy(
                            out_hbm.at[pl.ds(0, 1)].at[0],
                            stage[p].at[0],
                            gsem[p],
                        ).wait()

                    return c

                lax.fori_loop(0, _GROWS, row, 0)
                pltpu.make_async_copy(
                    out_hbm.at[pl.ds(0, 1)].at[0], stage[p].at[0], gsem[p]
                ).wait()
                pltpu.async_copy(
                    stage[p],
                    out_hbm.at[pl.ds(base + grow0, _GROWS)],
                    wsem[p],
                )
            return carry

        lax.fori_loop(0, npairs, pair, 0)
        for p in range(2):
            pltpu.make_async_copy(
                out_hbm.at[pl.ds(0, _GROWS)], stage[p], wsem[p]
            ).wait()

    return gather_kernel


def kernel(token_ids, weight):
    b, s = token_ids.shape
    _, dim = weight.shape
    ids = token_ids.astype(jnp.int32)
    return _make_gather(dim, b, s)(ids, weight)


# speedup vs baseline: 1.0832x; 1.0045x over previous
import functools

import jax
import jax.numpy as jnp
from jax import lax
from jax.experimental import layout as jex_layout
from jax.experimental import pallas as pl
from jax.experimental.pallas import tpu as pltpu
from jax.experimental.pallas import tpu_sc as plsc

_GROWS = 8
_SLAB = 128
_NW = 32


def _make_gather(dim: int, b: int, s: int):
    rows_w = b // _NW
    npairs = rows_w // (2 * _GROWS)
    pairs_per_slab = _SLAB // (2 * _GROWS)
    mesh = plsc.VectorSubcoreMesh(core_axis_name="c", subcore_axis_name="s")

    @functools.partial(
        pl.kernel,
        mesh=mesh,
        out_type=jax.ShapeDtypeStruct((b, s, dim), jnp.float32),
        compiler_params=pltpu.CompilerParams(use_tc_tiling_on_sc=True),
        scratch_types=[
            pltpu.VMEM((_SLAB, s), jnp.int32),
            pltpu.VMEM((_GROWS, s, dim), jnp.float32),
            pltpu.VMEM((_GROWS, s, dim), jnp.float32),
            pltpu.SemaphoreType.DMA,
            pltpu.SemaphoreType.DMA,
            pltpu.SemaphoreType.DMA,
            pltpu.SemaphoreType.DMA,
        ],
    )
    def gather_kernel(idx_hbm, table_hbm, out_hbm, idx_v, st0, st1, g0, g1, w0, w1):
        stage = (st0, st1)
        gsem = (g0, g1)
        wsem = (w0, w1)
        wid = lax.axis_index("s") * 2 + lax.axis_index("c")
        base = wid * rows_w

        def pair(g2, carry):
            @pl.when(g2 % pairs_per_slab == 0)
            def _():
                pltpu.sync_copy(
                    idx_hbm.at[pl.ds(base + (g2 // pairs_per_slab) * _SLAB, _SLAB)],
                    idx_v,
                )

            for p in range(2):
                grow0 = (g2 * 2 + p) * _GROWS
                srow0 = (g2 % pairs_per_slab) * 2 * _GROWS + p * _GROWS

                @pl.when(g2 > 0)
                def _():
                    pltpu.make_async_copy(
                        out_hbm.at[pl.ds(0, _GROWS)], stage[p], wsem[p]
                    ).wait()

                def row(r, c):
                    va = idx_v[srow0 + r, pl.ds(0, 16)]
                    vb = idx_v[srow0 + r, pl.ds(s - 16, 16)]
                    for t in range(s):
                        rid = va[t] if t < 16 else vb[t - (s - 16)]
                        pltpu.async_copy(
                            table_hbm.at[pl.ds(rid, 1)],
                            stage[p].at[r].at[pl.ds(t, 1)],
                            gsem[p],
                        )

                    @pl.when(r > 0)
                    def _():
                        pltpu.make_async_copy(
                            out_hbm.at[pl.ds(0, 1)].at[0],
                            stage[p].at[0],
                            gsem[p],
                        ).wait()

                    return c

                lax.fori_loop(0, _GROWS, row, 0)
                pltpu.make_async_copy(
                    out_hbm.at[pl.ds(0, 1)].at[0], stage[p].at[0], gsem[p]
                ).wait()
                pltpu.async_copy(
                    stage[p],
                    out_hbm.at[pl.ds(base + grow0, _GROWS)],
                    wsem[p],
                )
            return carry

        lax.fori_loop(0, npairs, pair, 0)
        for p in range(2):
            pltpu.make_async_copy(
                out_hbm.at[pl.ds(0, _GROWS)], stage[p], wsem[p]
            ).wait()

    return gather_kernel


def kernel(token_ids, weight):
    b, s = token_ids.shape
    _, dim = weight.shape
    ids = token_ids.astype(jnp.int32)
    wrm = jex_layout.with_layout_constraint(
        weight, jex_layout.Layout((1, 0), tiling=((8, 128),))
    )
    return _make_gather(dim, b, s)(ids, wrm)


# speedup vs baseline: 1.2912x; 1.1920x over previous
import functools

import jax
import jax.numpy as jnp
from jax import lax
from jax.experimental import layout as jex_layout
from jax.experimental import pallas as pl
from jax.experimental.pallas import tpu as pltpu
from jax.experimental.pallas import tpu_sc as plsc

_GROWS = 8
_SLAB = 128
_NW = 32


def _make_gather(dim: int, b: int, s: int):
    rows_w = b // _NW
    npairs = rows_w // (2 * _GROWS)
    pairs_per_slab = _SLAB // (2 * _GROWS)
    mesh = plsc.VectorSubcoreMesh(core_axis_name="c", subcore_axis_name="s")

    @functools.partial(
        pl.kernel,
        mesh=mesh,
        out_type=jax.ShapeDtypeStruct((b, s, dim), jnp.float32),
        compiler_params=pltpu.CompilerParams(use_tc_tiling_on_sc=True),
        scratch_types=[
            pltpu.VMEM((_SLAB, s), jnp.int32),
            pltpu.VMEM((_GROWS, s, dim), jnp.float32),
            pltpu.VMEM((_GROWS, s, dim), jnp.float32),
            pltpu.SemaphoreType.DMA,
            pltpu.SemaphoreType.DMA,
            pltpu.SemaphoreType.DMA,
            pltpu.SemaphoreType.DMA,
        ],
    )
    def gather_kernel(idx_hbm, table_hbm, out_hbm, idx_v, st0, st1, g0, g1, w0, w1):
        stage = (st0, st1)
        gsem = (g0, g1)
        wsem = (w0, w1)
        wid = lax.axis_index("s") * 2 + lax.axis_index("c")
        base = wid * rows_w

        def pair(g2, carry):
            @pl.when(g2 % pairs_per_slab == 0)
            def _():
                pltpu.sync_copy(
                    idx_hbm.at[pl.ds(base + (g2 // pairs_per_slab) * _SLAB, _SLAB)],
                    idx_v,
                )

            for p in range(2):
                grow0 = (g2 * 2 + p) * _GROWS
                srow0 = (g2 % pairs_per_slab) * 2 * _GROWS + p * _GROWS

                @pl.when(g2 > 0)
                def _():
                    pltpu.make_async_copy(
                        out_hbm.at[pl.ds(0, _GROWS)], stage[p], wsem[p]
                    ).wait()

                def row(r, c):
                    va = idx_v[srow0 + r, pl.ds(0, 16)]
                    vb = idx_v[srow0 + r, pl.ds(s - 16, 16)]
                    for t in range(s):
                        rid = va[t] if t < 16 else vb[t - (s - 16)]
                        pltpu.async_copy(
                            table_hbm.at[pl.ds(rid, 1)],
                            stage[p].at[r].at[pl.ds(t, 1)],
                            gsem[p],
                        )
                    return c

                lax.fori_loop(0, _GROWS, row, 0)
                pltpu.make_async_copy(
                    out_hbm.at[pl.ds(0, _GROWS)], stage[p], gsem[p]
                ).wait()
                pltpu.async_copy(
                    stage[p],
                    out_hbm.at[pl.ds(base + grow0, _GROWS)],
                    wsem[p],
                )
            return carry

        lax.fori_loop(0, npairs, pair, 0)
        for p in range(2):
            pltpu.make_async_copy(
                out_hbm.at[pl.ds(0, _GROWS)], stage[p], wsem[p]
            ).wait()

    return gather_kernel


def kernel(token_ids, weight):
    b, s = token_ids.shape
    _, dim = weight.shape
    ids = token_ids.astype(jnp.int32)
    wrm = jex_layout.with_layout_constraint(
        weight, jex_layout.Layout((1, 0), tiling=((8, 128),))
    )
    return _make_gather(dim, b, s)(ids, wrm)


# speedup vs baseline: 1.3285x; 1.0289x over previous
import functools

import jax
import jax.numpy as jnp
from jax import lax
from jax.experimental import layout as jex_layout
from jax.experimental import pallas as pl
from jax.experimental.pallas import tpu as pltpu
from jax.experimental.pallas import tpu_sc as plsc

_GROWS = 16
_SLAB = 128
_NW = 32


def _make_gather(dim: int, b: int, s: int):
    rows_w = b // _NW
    npairs = rows_w // (2 * _GROWS)
    pairs_per_slab = _SLAB // (2 * _GROWS)
    mesh = plsc.VectorSubcoreMesh(core_axis_name="c", subcore_axis_name="s")

    @functools.partial(
        pl.kernel,
        mesh=mesh,
        out_type=jax.ShapeDtypeStruct((b, s, dim), jnp.float32),
        compiler_params=pltpu.CompilerParams(use_tc_tiling_on_sc=True),
        scratch_types=[
            pltpu.VMEM((_SLAB, s), jnp.int32),
            pltpu.VMEM((_GROWS, s, dim), jnp.float32),
            pltpu.VMEM((_GROWS, s, dim), jnp.float32),
            pltpu.SemaphoreType.DMA,
            pltpu.SemaphoreType.DMA,
            pltpu.SemaphoreType.DMA,
            pltpu.SemaphoreType.DMA,
        ],
    )
    def gather_kernel(idx_hbm, table_hbm, out_hbm, idx_v, st0, st1, g0, g1, w0, w1):
        stage = (st0, st1)
        gsem = (g0, g1)
        wsem = (w0, w1)
        wid = lax.axis_index("s") * 2 + lax.axis_index("c")
        base = wid * rows_w

        def pair(g2, carry):
            @pl.when(g2 % pairs_per_slab == 0)
            def _():
                pltpu.sync_copy(
                    idx_hbm.at[pl.ds(base + (g2 // pairs_per_slab) * _SLAB, _SLAB)],
                    idx_v,
                )

            for p in range(2):
                grow0 = (g2 * 2 + p) * _GROWS
                srow0 = (g2 % pairs_per_slab) * 2 * _GROWS + p * _GROWS

                @pl.when(g2 > 0)
                def _():
                    pltpu.make_async_copy(
                        out_hbm.at[pl.ds(0, _GROWS)], stage[p], wsem[p]
                    ).wait()

                def row(r, c):
                    va = idx_v[srow0 + r, pl.ds(0, 16)]
                    vb = idx_v[srow0 + r, pl.ds(s - 16, 16)]
                    for t in range(s):
                        rid = va[t] if t < 16 else vb[t - (s - 16)]
                        pltpu.async_copy(
                            table_hbm.at[pl.ds(rid, 1)],
                            stage[p].at[r].at[pl.ds(t, 1)],
                            gsem[p],
                        )
                    return c

                lax.fori_loop(0, _GROWS, row, 0)
                pltpu.make_async_copy(
                    out_hbm.at[pl.ds(0, _GROWS)], stage[p], gsem[p]
                ).wait()
                pltpu.async_copy(
                    stage[p],
                    out_hbm.at[pl.ds(base + grow0, _GROWS)],
                    wsem[p],
                )
            return carry

        lax.fori_loop(0, npairs, pair, 0)
        for p in range(2):
            pltpu.make_async_copy(
                out_hbm.at[pl.ds(0, _GROWS)], stage[p], wsem[p]
            ).wait()

    return gather_kernel


def kernel(token_ids, weight):
    b, s = token_ids.shape
    _, dim = weight.shape
    ids = token_ids.astype(jnp.int32)
    wrm = jex_layout.with_layout_constraint(
        weight, jex_layout.Layout((1, 0), tiling=((8, 128),))
    )
    return _make_gather(dim, b, s)(ids, wrm)


# speedup vs baseline: 1.3299x; 1.0011x over previous
import functools

import jax
import jax.numpy as jnp
from jax import lax
from jax.experimental import pallas as pl
from jax.experimental.pallas import tpu as pltpu
from jax.experimental.pallas import tpu_sc as plsc

_GROWS = 16
_SLAB = 128
_NW = 32


def _make_gather(dim: int, b: int, s: int):
    rows_w = b // _NW
    npairs = rows_w // (2 * _GROWS)
    pairs_per_slab = _SLAB // (2 * _GROWS)
    mesh = plsc.VectorSubcoreMesh(core_axis_name="c", subcore_axis_name="s")

    @functools.partial(
        pl.kernel,
        mesh=mesh,
        out_type=jax.ShapeDtypeStruct((b, s, dim), jnp.float32),
        compiler_params=pltpu.CompilerParams(use_tc_tiling_on_sc=True),
        scratch_types=[
            pltpu.VMEM((_SLAB, s), jnp.int32),
            pltpu.VMEM((_GROWS, s, dim), jnp.float32),
            pltpu.VMEM((_GROWS, s, dim), jnp.float32),
            pltpu.SemaphoreType.DMA,
            pltpu.SemaphoreType.DMA,
            pltpu.SemaphoreType.DMA,
            pltpu.SemaphoreType.DMA,
        ],
    )
    def gather_kernel(idx_hbm, table_hbm, out_hbm, idx_v, st0, st1, g0, g1, w0, w1):
        stage = (st0, st1)
        gsem = (g0, g1)
        wsem = (w0, w1)
        wid = lax.axis_index("s") * 2 + lax.axis_index("c")
        base = wid * rows_w

        def pair(g2, carry):
            @pl.when(g2 % pairs_per_slab == 0)
            def _():
                pltpu.sync_copy(
                    idx_hbm.at[pl.ds(base + (g2 // pairs_per_slab) * _SLAB, _SLAB)],
                    idx_v,
                )

            for p in range(2):
                grow0 = (g2 * 2 + p) * _GROWS
                srow0 = (g2 % pairs_per_slab) * 2 * _GROWS + p * _GROWS

                @pl.when(g2 > 0)
                def _():
                    pltpu.make_async_copy(
                        out_hbm.at[pl.ds(0, _GROWS)], stage[p], wsem[p]
                    ).wait()

                def row(r, c):
                    va = idx_v[srow0 + r, pl.ds(0, 16)]
                    vb = idx_v[srow0 + r, pl.ds(s - 16, 16)]
                    for t in range(s):
                        rid = va[t] if t < 16 else vb[t - (s - 16)]
                        pltpu.async_copy(
                            table_hbm.at[pl.ds(rid, 1)],
                            stage[p].at[r].at[pl.ds(t, 1)],
                            gsem[p],
                        )
                    return c

                lax.fori_loop(0, _GROWS, row, 0)
                pltpu.make_async_copy(
                    out_hbm.at[pl.ds(0, _GROWS)], stage[p], gsem[p]
                ).wait()
                pltpu.async_copy(
                    stage[p],
                    out_hbm.at[pl.ds(base + grow0, _GROWS)],
                    wsem[p],
                )
            return carry

        lax.fori_loop(0, npairs, pair, 0)
        for p in range(2):
            pltpu.make_async_copy(
                out_hbm.at[pl.ds(0, _GROWS)], stage[p], wsem[p]
            ).wait()

    return gather_kernel


def kernel(token_ids, weight):
    b, s = token_ids.shape
    _, dim = weight.shape
    ids = token_ids.astype(jnp.int32)
    return _make_gather(dim, b, s)(ids, weight)
